# baseline (speedup 1.0000x reference)
import jax
import jax.numpy as jnp
from jax.experimental import pallas as pl


def _body(x_ref, t_ref, w_ref, b_ref, o_ref):
    xb = x_ref[...].astype(jnp.bfloat16)
    tb = t_ref[...]
    T = w_ref.shape[0]
    acc = None
    for t in range(T):
        p = jnp.dot(xb, w_ref[t], preferred_element_type=jnp.float32)
        p = p + b_ref[t][None, :]
        if acc is None:
            acc = p
        else:
            acc = jnp.where(tb == t, p, acc)
    o_ref[...] = acc


def kernel(x, node_types, W, b):
    N, D = x.shape
    T, H, _ = W.shape
    B = 1000
    assert N % B == 0
    nt2 = node_types.astype(jnp.int32).reshape(N, 1)
    Wt = jnp.swapaxes(W, 1, 2).astype(jnp.bfloat16)
    return pl.pallas_call(
        _body,
        grid=(N // B,),
        in_specs=[
            pl.BlockSpec((B, D), lambda i: (i, 0)),
            pl.BlockSpec((B, 1), lambda i: (i, 0)),
            pl.BlockSpec((T, D, H), lambda i: (0, 0, 0)),
            pl.BlockSpec((T, H), lambda i: (0, 0)),
        ],
        out_specs=pl.BlockSpec((B, H), lambda i: (i, 0)),
        out_shape=jax.ShapeDtypeStruct((N, H), x.dtype),
    )(x, nt2, Wt, b)


# speedup vs baseline: 1.2141x; 1.2141x over previous
import jax
import jax.numpy as jnp
from jax.experimental import pallas as pl


def _body(x_ref, t_ref, w_ref, b_ref, o_ref):
    xb = x_ref[...].astype(jnp.bfloat16)
    tb = t_ref[...]
    H = o_ref.shape[1]
    T = w_ref.shape[1] // H
    p = jnp.dot(xb, w_ref[...], preferred_element_type=jnp.float32)
    acc = p[:, 0:H] + b_ref[0][None, :]
    for t in range(1, T):
        acc = jnp.where(tb == t, p[:, t * H:(t + 1) * H] + b_ref[t][None, :], acc)
    o_ref[...] = acc


def kernel(x, node_types, W, b):
    N, D = x.shape
    T, H, _ = W.shape
    B = 2000
    assert N % B == 0
    nt2 = node_types.astype(jnp.int32).reshape(N, 1)
    Wc = jnp.swapaxes(W, 1, 2).transpose(1, 0, 2).reshape(D, T * H).astype(jnp.bfloat16)
    return pl.pallas_call(
        _body,
        grid=(N // B,),
        in_specs=[
            pl.BlockSpec((B, D), lambda i: (i, 0)),
            pl.BlockSpec((B, 1), lambda i: (i, 0)),
            pl.BlockSpec((D, T * H), lambda i: (0, 0)),
            pl.BlockSpec((T, H), lambda i: (0, 0)),
        ],
        out_specs=pl.BlockSpec((B, H), lambda i: (i, 0)),
        out_shape=jax.ShapeDtypeStruct((N, H), x.dtype),
    )(x, nt2, Wc, b)
